# baseline (speedup 1.0000x reference)
import math
import functools

import jax
import jax.numpy as jnp
import numpy as np
from jax import lax
from jax.experimental import pallas as pl
from jax.experimental.pallas import tpu as pltpu
from jax.experimental.pallas import tpu_sc as plsc

DIM = 64
MAX_LEN = 5000
SQRT_DIM = math.sqrt(DIM)

LANES = 16
NWORKERS = 32
SBLK = 128
NBUF = 2


def _make_pe_t(seq_len: int) -> np.ndarray:
    position = np.arange(0, MAX_LEN, dtype=np.float64)[:, None]
    div_term = np.exp(
        np.arange(0, DIM, 2, dtype=np.float64) * -(math.log(10000.0) / DIM)
    )
    pe = np.zeros((MAX_LEN, DIM), dtype=np.float64)
    pe[:, 0::2] = np.sin(position * div_term)
    pe[:, 1::2] = np.cos(position * div_term)
    return np.ascontiguousarray(pe[:seq_len].T).astype(np.float32)


@functools.partial(jax.jit, static_argnames=("S", "B"))
def _embed_sc(idx_t, W2, pe_t, *, S, B):
    n_sblk = S // SBLK
    b_half = B * n_sblk // NWORKERS
    n_groups = SBLK // LANES

    mesh = plsc.VectorSubcoreMesh(core_axis_name="core",
                                  subcore_axis_name="subcore")

    @pl.kernel(
        out_type=jax.ShapeDtypeStruct((B, DIM, S), jnp.float32),
        mesh=mesh,
        compiler_params=pltpu.CompilerParams(use_tc_tiling_on_sc=True,
                                             needs_layout_passes=False),
        scratch_types=[
            pltpu.VMEM((b_half, SBLK), jnp.int32),
            pltpu.VMEM((DIM, SBLK), jnp.float32),
            pltpu.VMEM((NBUF, SBLK), jnp.int32),
            pltpu.VMEM((NBUF, SBLK, SBLK), jnp.float32),
            pltpu.VMEM((NBUF, DIM, SBLK), jnp.float32),
            pltpu.SemaphoreType.DMA,
            pltpu.SemaphoreType.DMA((NBUF,)),
            pltpu.SemaphoreType.DMA((NBUF,)),
        ],
    )
    def kernel_fn(W2_hbm, i_hbm, pe_hbm, o_hbm,
                  idx_v, pe_v, idxp_v, buf_v, out_v, sem_in, sem_g, sem_s):
        w = lax.axis_index("core") * 16 + lax.axis_index("subcore")
        sblk = w // 2
        b0 = (w % 2) * b_half
        s0 = sblk * SBLK

        c_idx = pltpu.async_copy(
            i_hbm.at[pl.ds(b0, b_half), pl.ds(s0, SBLK)], idx_v, sem_in)
        c_pe = pltpu.async_copy(pe_hbm.at[:, pl.ds(s0, SBLK)], pe_v, sem_in)
        c_idx.wait()
        c_pe.wait()

        def prep_idx(c, slot):
            for g in range(n_groups):
                sl = pl.ds(g * LANES, LANES)
                idxp_v[slot, sl] = lax.shift_right_logical(idx_v[c, sl], 1)

        def gather_copy(slot):
            return pltpu.make_async_copy(
                W2_hbm.at[idxp_v.at[slot]], buf_v.at[slot], sem_g.at[slot])

        def compute(c, slot):
            for g in range(n_groups):
                sl = pl.ds(g * LANES, LANES)
                rowv = jax.lax.iota(jnp.int32, LANES) + (g * LANES)
                par64 = lax.shift_left(
                    lax.bitwise_and(idx_v[c, sl], 1), 6)

                @pl.loop(0, DIM)
                def _(d, sl=sl, rowv=rowv, par64=par64):
                    vals = plsc.load_gather(buf_v.at[slot], [rowv, par64 + d])
                    out_v[slot, d, sl] = vals * SQRT_DIM + pe_v[d, sl]

        def writeback_copy(c, slot):
            return pltpu.make_async_copy(
                out_v.at[slot],
                o_hbm.at[b0 + c, :, pl.ds(s0, SBLK)],
                sem_s.at[slot])

        prep_idx(0, 0)
        gather_copy(0).start()

        @pl.loop(0, b_half, step=NBUF)
        def _(c0):
            for u in range(NBUF):
                c = c0 + u
                slot = u
                nslot = (u + 1) % NBUF

                @pl.when(c + 1 < b_half)
                def _(c=c, nslot=nslot):
                    prep_idx(c + 1, nslot)
                    gather_copy(nslot).start()

                gather_copy(slot).wait()

                @pl.when(c >= NBUF)
                def _(c=c, slot=slot):
                    writeback_copy(c - NBUF, slot).wait()

                compute(c, slot)
                writeback_copy(c, slot).start()

        for u in range(NBUF):
            writeback_copy(b_half - NBUF + u, u).wait()

    return kernel_fn(W2, idx_t, pe_t)


def kernel(input, W):
    S, B, _ = input.shape
    idx_t = jnp.transpose(input[..., 0])
    W2 = jnp.reshape(W, (W.shape[0] // 2, 2 * DIM))
    pe_t = jnp.asarray(_make_pe_t(S))
    out_t = _embed_sc(idx_t, W2, pe_t, S=S, B=B)
    return jnp.transpose(out_t, (2, 0, 1))
